# baseline (speedup 1.0000x reference)
import functools

import jax
import jax.numpy as jnp
from jax import lax
from jax.experimental import pallas as pl
from jax.experimental.pallas import tpu as pltpu
from jax.experimental.pallas import tpu_sc as plsc

_RES = 32
_F = 256


def _pos_embed_body(row_hbm, col_hbm, out_hbm):
    nc = 2
    wid = lax.axis_index("s") * nc + lax.axis_index("c")
    base = wid * _RES

    pltpu.sync_copy(col_hbm, out_hbm.at[pl.ds(base, _RES), pl.ds(0, _F)])

    pltpu.sync_copy(row_hbm.at[wid], out_hbm.at[base, pl.ds(_F, _F)])
    n = 1
    while n < _RES:
        pltpu.sync_copy(out_hbm.at[pl.ds(base, n), pl.ds(_F, _F)],
                        out_hbm.at[pl.ds(base + n, n), pl.ds(_F, _F)])
        n *= 2


@functools.partial(jax.jit)
def _pos_embed(row_embed, col_embed):
    mesh = plsc.VectorSubcoreMesh(core_axis_name="c", subcore_axis_name="s")
    k = functools.partial(
        pl.kernel,
        mesh=mesh,
        out_type=jax.ShapeDtypeStruct((_RES * _RES, 2 * _F), jnp.float32),
    )(_pos_embed_body)
    return k(row_embed, col_embed)


def kernel(row_embed, col_embed):
    pos = _pos_embed(row_embed, col_embed)
    return pos[None, :, :]


# speedup vs baseline: 3.3664x; 3.3664x over previous
import functools

import jax
import jax.numpy as jnp
from jax import lax
from jax.experimental import pallas as pl
from jax.experimental.pallas import tpu as pltpu
from jax.experimental.pallas import tpu_sc as plsc

_RES = 32
_F = 256


def _pos_embed_body(row_hbm, col_hbm, out_hbm, out_v, row_v, sem_c, sem_r):
    nc = 2
    wid = lax.axis_index("s") * nc + lax.axis_index("c")
    base = wid * _RES

    cp_col = pltpu.make_async_copy(col_hbm, out_v.at[:, pl.ds(0, _F)], sem_c)
    cp_row = pltpu.make_async_copy(row_hbm.at[wid], row_v, sem_r)
    cp_col.start()
    cp_row.start()
    cp_row.wait()

    for k in range(_F // 16):
        v = row_v[pl.ds(k * 16, 16)]
        for t in range(_RES):
            out_v[t, pl.ds(_F + k * 16, 16)] = v

    cp_col.wait()
    pltpu.sync_copy(out_v, out_hbm.at[pl.ds(base, _RES)])


@functools.partial(jax.jit)
def _pos_embed(row_embed, col_embed):
    mesh = plsc.VectorSubcoreMesh(core_axis_name="c", subcore_axis_name="s")
    k = functools.partial(
        pl.kernel,
        mesh=mesh,
        out_type=jax.ShapeDtypeStruct((_RES * _RES, 2 * _F), jnp.float32),
        scratch_types=[
            pltpu.VMEM((_RES, 2 * _F), jnp.float32),
            pltpu.VMEM((_F,), jnp.float32),
            pltpu.SemaphoreType.DMA,
            pltpu.SemaphoreType.DMA,
        ],
    )(_pos_embed_body)
    return k(row_embed, col_embed)


def kernel(row_embed, col_embed):
    pos = _pos_embed(row_embed, col_embed)
    return pos[None, :, :]


# speedup vs baseline: 7.5170x; 2.2329x over previous
import functools

import jax
import jax.numpy as jnp
from jax.experimental import pallas as pl

_RES = 32
_F = 256


def _pos_embed_body(row_ref, col_ref, out_ref):
    r = pl.program_id(0)
    out_ref[:, 0:_F] = col_ref[...]
    out_ref[:, _F:2 * _F] = jnp.broadcast_to(row_ref[pl.ds(r, 1), :], (_RES, _F))


@jax.jit
def _pos_embed(row_embed, col_embed):
    return pl.pallas_call(
        _pos_embed_body,
        grid=(_RES,),
        in_specs=[
            pl.BlockSpec((_RES, _F), lambda r: (0, 0)),
            pl.BlockSpec((_RES, _F), lambda r: (0, 0)),
        ],
        out_specs=pl.BlockSpec((_RES, 2 * _F), lambda r: (r, 0)),
        out_shape=jax.ShapeDtypeStruct((_RES * _RES, 2 * _F), jnp.float32),
    )(row_embed, col_embed)


def kernel(row_embed, col_embed):
    pos = _pos_embed(row_embed, col_embed)
    return pos[None, :, :]


# speedup vs baseline: 20.9351x; 2.7851x over previous
import functools

import jax
import jax.numpy as jnp
from jax.experimental import pallas as pl

_RES = 32
_F = 256


_RPB = 4


def _pos_embed_body(row_ref, col_ref, out_ref):
    g = pl.program_id(0)
    col = col_ref[...]
    for i in range(_RPB):
        out_ref[pl.ds(i * _RES, _RES), 0:_F] = col
        out_ref[pl.ds(i * _RES, _RES), _F:2 * _F] = jnp.broadcast_to(
            row_ref[pl.ds(g * _RPB + i, 1), :], (_RES, _F))


@jax.jit
def _pos_embed(row_embed, col_embed):
    return pl.pallas_call(
        _pos_embed_body,
        grid=(_RES // _RPB,),
        in_specs=[
            pl.BlockSpec((_RES, _F), lambda g: (0, 0)),
            pl.BlockSpec((_RES, _F), lambda g: (0, 0)),
        ],
        out_specs=pl.BlockSpec((_RPB * _RES, 2 * _F), lambda g: (g, 0)),
        out_shape=jax.ShapeDtypeStruct((_RES * _RES, 2 * _F), jnp.float32),
    )(row_embed, col_embed)


def kernel(row_embed, col_embed):
    pos = _pos_embed(row_embed, col_embed)
    return pos[None, :, :]


# speedup vs baseline: 29.9647x; 1.4313x over previous
import functools

import jax
import jax.numpy as jnp
from jax.experimental import pallas as pl

_RES = 32
_F = 256


_RPB = 8


def _pos_embed_body(row_ref, col_ref, out_ref):
    g = pl.program_id(0)
    col = col_ref[...]
    for i in range(_RPB):
        out_ref[pl.ds(i * _RES, _RES), 0:_F] = col
        out_ref[pl.ds(i * _RES, _RES), _F:2 * _F] = jnp.broadcast_to(
            row_ref[pl.ds(g * _RPB + i, 1), :], (_RES, _F))


@jax.jit
def _pos_embed(row_embed, col_embed):
    return pl.pallas_call(
        _pos_embed_body,
        grid=(_RES // _RPB,),
        in_specs=[
            pl.BlockSpec((_RES, _F), lambda g: (0, 0)),
            pl.BlockSpec((_RES, _F), lambda g: (0, 0)),
        ],
        out_specs=pl.BlockSpec((_RPB * _RES, 2 * _F), lambda g: (g, 0)),
        out_shape=jax.ShapeDtypeStruct((_RES * _RES, 2 * _F), jnp.float32),
    )(row_embed, col_embed)


def kernel(row_embed, col_embed):
    pos = _pos_embed(row_embed, col_embed)
    return pos[None, :, :]


# speedup vs baseline: 35.3807x; 1.1807x over previous
import functools

import jax
import jax.numpy as jnp
from jax.experimental import pallas as pl

_RES = 32
_F = 256


_RPB = 16


def _pos_embed_body(row_ref, col_ref, out_ref):
    g = pl.program_id(0)
    col = col_ref[...]
    for i in range(_RPB):
        out_ref[pl.ds(i * _RES, _RES), 0:_F] = col
        out_ref[pl.ds(i * _RES, _RES), _F:2 * _F] = jnp.broadcast_to(
            row_ref[pl.ds(g * _RPB + i, 1), :], (_RES, _F))


@jax.jit
def _pos_embed(row_embed, col_embed):
    return pl.pallas_call(
        _pos_embed_body,
        grid=(_RES // _RPB,),
        in_specs=[
            pl.BlockSpec((_RES, _F), lambda g: (0, 0)),
            pl.BlockSpec((_RES, _F), lambda g: (0, 0)),
        ],
        out_specs=pl.BlockSpec((_RPB * _RES, 2 * _F), lambda g: (g, 0)),
        out_shape=jax.ShapeDtypeStruct((_RES * _RES, 2 * _F), jnp.float32),
    )(row_embed, col_embed)


def kernel(row_embed, col_embed):
    pos = _pos_embed(row_embed, col_embed)
    return pos[None, :, :]
